# baseline (speedup 1.0000x reference)
import functools

import jax
import jax.numpy as jnp
from jax import lax
from jax.experimental import pallas as pl
from jax.experimental.pallas import tpu as pltpu
from jax.experimental.pallas import tpu_sc as plsc

N = 10000
E = 320000
D = 128
DE = 16
EMB = D + DE

NC = 2
NS = 16
NW = NC * NS
EPW = E // NW
CHUNK = 80
NCHUNK = EPW // CHUNK

EB = 2560
NEB = E // EB

ZCH = 80
NZCH = N // ZCH
ZITER = (NZCH + NS - 1) // NS


def _sc_mesh():
    return plsc.VectorSubcoreMesh(core_axis_name="c", subcore_axis_name="s")


@functools.cache
def _sc_gather(epw, chunk, nchunk):
    def body(x_hbm, idx_hbm, out_hbm, idx_v, rows_v, gsem, ssem):
        wid = lax.axis_index("s") * NC + lax.axis_index("c")
        pltpu.sync_copy(idx_hbm.at[wid], idx_v)
        base = wid * epw

        def g_desc(j, b):
            return pltpu.make_async_copy(
                x_hbm.at[idx_v.at[j]], rows_v.at[b], gsem.at[b])

        def s_desc(j, b):
            return pltpu.make_async_copy(
                rows_v.at[b], out_hbm.at[pl.ds(base + j * chunk, chunk)],
                ssem.at[b])

        g_desc(0, 0).start()

        def step(j, _):
            b = j % 2
            nb = 1 - b
            g_desc(j, b).wait()

            @pl.when(j + 1 < nchunk)
            def _():
                @pl.when(j >= 1)
                def _():
                    s_desc(j - 1, nb).wait()
                g_desc(j + 1, nb).start()

            s_desc(j, b).start()
            return 0

        lax.fori_loop(0, nchunk, step, 0)
        bl = (nchunk - 1) % 2
        s_desc(nchunk - 2, 1 - bl).wait()
        s_desc(nchunk - 1, bl).wait()

    return pl.kernel(
        body,
        out_type=jax.ShapeDtypeStruct((epw * NW, D), jnp.float32),
        mesh=_sc_mesh(),
        scratch_types=[
            pltpu.VMEM((nchunk, chunk), jnp.int32),
            pltpu.VMEM((2, chunk, D), jnp.float32),
            pltpu.SemaphoreType.DMA((2,)),
            pltpu.SemaphoreType.DMA((2,)),
        ],
    )


@functools.cache
def _sc_scatter(epw, chunk, nchunk):
    def body(msg_hbm, idx_hbm, out_hbm, idx_v, rows_v, hacc, lsem, asem):
        cid = lax.axis_index("c")
        sid = lax.axis_index("s")
        wid = sid * NC + cid

        def zrow(i, _):
            def zseg(k, _):
                rows_v[0, i, pl.ds(k * 16, 16)] = jnp.zeros(
                    (16,), jnp.float32)
                return 0
            lax.fori_loop(0, D // 16, zseg, 0)
            return 0

        lax.fori_loop(0, ZCH, zrow, 0)

        def zcopy(t, _):
            j = sid + t * NS

            @pl.when(j < NZCH)
            def _():
                pltpu.sync_copy(
                    rows_v.at[0].at[pl.ds(0, ZCH)],
                    hacc.at[pl.ds(j * ZCH, ZCH)])
            return 0

        lax.fori_loop(0, ZITER, zcopy, 0)
        plsc.subcore_barrier()

        pltpu.sync_copy(idx_hbm.at[wid], idx_v)
        base = wid * epw

        def l_desc(j, b):
            return pltpu.make_async_copy(
                msg_hbm.at[pl.ds(base + j * chunk, chunk)],
                rows_v.at[b].at[pl.ds(0, chunk)], lsem.at[b])

        def a_desc(j, b):
            return pltpu.make_async_copy(
                rows_v.at[b].at[pl.ds(0, chunk)], hacc.at[idx_v.at[j]],
                asem.at[b])

        l_desc(0, 0).start()

        def step(j, _):
            b = j % 2
            nb = 1 - b
            l_desc(j, b).wait()

            @pl.when(j + 1 < nchunk)
            def _():
                @pl.when(j >= 1)
                def _():
                    a_desc(j - 1, nb).wait()
                l_desc(j + 1, nb).start()

            a_desc(j, b).start(add=True)
            return 0

        lax.fori_loop(0, nchunk, step, 0)
        bl = (nchunk - 1) % 2
        a_desc(nchunk - 2, 1 - bl).wait()
        a_desc(nchunk - 1, bl).wait()
        plsc.subcore_barrier()

        def ecopy(t, _):
            j = sid + t * NS

            @pl.when(j < NZCH)
            def _():
                sl = pl.ds(j * ZCH, ZCH)
                pltpu.sync_copy(hacc.at[sl], out_hbm.at[cid].at[sl])
            return 0

        lax.fori_loop(0, ZITER, ecopy, 0)

    buf_rows = max(chunk, ZCH)
    return pl.kernel(
        body,
        out_type=jax.ShapeDtypeStruct((NC, N, D), jnp.float32),
        mesh=_sc_mesh(),
        scratch_types=[
            pltpu.VMEM((nchunk, chunk), jnp.int32),
            pltpu.VMEM((2, buf_rows, D), jnp.float32),
            pltpu.VMEM_SHARED((N, D), jnp.float32),
            pltpu.SemaphoreType.DMA((2,)),
            pltpu.SemaphoreType.DMA((2,)),
        ],
    )


def _stats_body(xg_ref, a_ref, w1x_ref, w1a_ref, b1_ref, out_ref, e_ref):
    i = pl.program_id(0)
    a = a_ref[...]
    e_ref[...] = jnp.maximum(a, 0.0)
    xgb = xg_ref[...].astype(jnp.bfloat16)
    z = jnp.dot(xgb, w1x_ref[...], preferred_element_type=jnp.float32)
    z = z + jnp.dot(a.astype(jnp.bfloat16), w1a_ref[...],
                    preferred_element_type=jnp.float32)
    z = z + b1_ref[...]
    s1 = jnp.sum(z, axis=0, keepdims=True)
    s2 = jnp.sum(z * z, axis=0, keepdims=True)
    blk = jnp.concatenate(
        [s1, s2, jnp.zeros((6, EMB), jnp.float32)], axis=0)

    @pl.when(i == 0)
    def _():
        out_ref[...] = blk

    @pl.when(i > 0)
    def _():
        out_ref[...] = out_ref[...] + blk


def _tc_stats(xg, a, w1x, w1a, b1r):
    return pl.pallas_call(
        _stats_body,
        grid=(NEB,),
        in_specs=[
            pl.BlockSpec((EB, D), lambda i: (i, 0)),
            pl.BlockSpec((EB, DE), lambda i: (i, 0)),
            pl.BlockSpec((D, EMB), lambda i: (0, 0)),
            pl.BlockSpec((DE, EMB), lambda i: (0, 0)),
            pl.BlockSpec((1, EMB), lambda i: (0, 0)),
        ],
        out_specs=[
            pl.BlockSpec((8, EMB), lambda i: (0, 0)),
            pl.BlockSpec((EB, DE), lambda i: (i, 0)),
        ],
        out_shape=[
            jax.ShapeDtypeStruct((8, EMB), jnp.float32),
            jax.ShapeDtypeStruct((E, DE), jnp.float32),
        ],
        compiler_params=pltpu.CompilerParams(
            dimension_semantics=("arbitrary",)),
    )(xg, a, w1x, w1a, b1r)


def _main_body(st_ref, xg_ref, a_ref, w1x_ref, w1a_ref, b1_ref,
               g_ref, be_ref, w2_ref, b2_ref, msg_ref):
    st = st_ref[...]
    mu = st[0:1, :] * (1.0 / E)
    ex2 = st[1:2, :] * (1.0 / E)
    var = ex2 - mu * mu
    scale = lax.rsqrt(var + 1e-5) * g_ref[...]
    xgb = xg_ref[...].astype(jnp.bfloat16)
    z = jnp.dot(xgb, w1x_ref[...], preferred_element_type=jnp.float32)
    z = z + jnp.dot(a_ref[...].astype(jnp.bfloat16), w1a_ref[...],
                    preferred_element_type=jnp.float32)
    z = z + b1_ref[...]
    zn = (z - mu) * scale + be_ref[...]
    r = jnp.maximum(zn, 0.0).astype(jnp.bfloat16)
    msg_ref[...] = jnp.dot(r, w2_ref[...],
                           preferred_element_type=jnp.float32) + b2_ref[...]


def _tc_main(st, xg, a, w1x, w1a, b1r, gr, br, w2, b2r):
    return pl.pallas_call(
        _main_body,
        grid=(NEB,),
        in_specs=[
            pl.BlockSpec((8, EMB), lambda i: (0, 0)),
            pl.BlockSpec((EB, D), lambda i: (i, 0)),
            pl.BlockSpec((EB, DE), lambda i: (i, 0)),
            pl.BlockSpec((D, EMB), lambda i: (0, 0)),
            pl.BlockSpec((DE, EMB), lambda i: (0, 0)),
            pl.BlockSpec((1, EMB), lambda i: (0, 0)),
            pl.BlockSpec((1, EMB), lambda i: (0, 0)),
            pl.BlockSpec((1, EMB), lambda i: (0, 0)),
            pl.BlockSpec((EMB, D), lambda i: (0, 0)),
            pl.BlockSpec((1, D), lambda i: (0, 0)),
        ],
        out_specs=pl.BlockSpec((EB, D), lambda i: (i, 0)),
        out_shape=jax.ShapeDtypeStruct((E, D), jnp.float32),
        compiler_params=pltpu.CompilerParams(
            dimension_semantics=("arbitrary",)),
    )(st, xg, a, w1x, w1a, b1r, gr, br, w2, b2r)


def _final_body(p_ref, h_ref):
    h_ref[...] = jnp.maximum(p_ref[0] + p_ref[1], 0.0)


def _tc_final(partials):
    nb = 2000
    return pl.pallas_call(
        _final_body,
        grid=(N // nb,),
        in_specs=[pl.BlockSpec((NC, nb, D), lambda i: (0, i, 0))],
        out_specs=pl.BlockSpec((nb, D), lambda i: (i, 0)),
        out_shape=jax.ShapeDtypeStruct((N, D), jnp.float32),
    )(partials)


def kernel(x, edge_index, edge_attr, W1, b1, gamma, beta, W2, b2):
    src3d = edge_index[0].reshape(NW, NCHUNK, CHUNK)
    dst3d = edge_index[1].reshape(NW, NCHUNK, CHUNK)
    w1x = W1[:D]
    w1a = W1[D:]
    b1r = b1.reshape(1, EMB)
    gr = gamma.reshape(1, EMB)
    br = beta.reshape(1, EMB)
    b2r = b2.reshape(1, D)

    w1xb = w1x.astype(jnp.bfloat16)
    w1ab = w1a.astype(jnp.bfloat16)
    w2b = W2.astype(jnp.bfloat16)
    xg = _sc_gather(EPW, CHUNK, NCHUNK)(x, src3d)
    st, e_out = _tc_stats(xg, edge_attr, w1xb, w1ab, b1r)
    msg = _tc_main(st, xg, edge_attr, w1xb, w1ab, b1r, gr, br, w2b, b2r)
    partials = _sc_scatter(EPW, CHUNK, NCHUNK)(msg, dst3d)
    h = _tc_final(partials)
    return (h, e_out)


# speedup vs baseline: 1.0885x; 1.0885x over previous
import functools

import jax
import jax.numpy as jnp
from jax import lax
from jax.experimental import pallas as pl
from jax.experimental.pallas import tpu as pltpu
from jax.experimental.pallas import tpu_sc as plsc

N = 10000
E = 320000
D = 128
DE = 16
EMB = D + DE

NC = 2
NS = 16
NW = NC * NS
EPW = E // NW
CHUNK = 80
NCHUNK = EPW // CHUNK

EB = 2560
NEB = E // EB

ZCH = 80
NZCH = N // ZCH
ZITER = (NZCH + NS - 1) // NS


def _sc_mesh():
    return plsc.VectorSubcoreMesh(core_axis_name="c", subcore_axis_name="s")


@functools.cache
def _sc_gather(epw, chunk, nchunk):
    def body(x_hbm, idx_hbm, out_hbm, idx_v, rows_v, gsem, ssem):
        wid = lax.axis_index("s") * NC + lax.axis_index("c")
        pltpu.sync_copy(idx_hbm.at[wid], idx_v)
        base = wid * epw

        def g_desc(j, b):
            return pltpu.make_async_copy(
                x_hbm.at[idx_v.at[j]], rows_v.at[b], gsem.at[b])

        def s_desc(j, b):
            return pltpu.make_async_copy(
                rows_v.at[b], out_hbm.at[pl.ds(base + j * chunk, chunk)],
                ssem.at[b])

        g_desc(0, 0).start()
        g_desc(1, 1).start()

        def step(j, _):
            b = j % 3
            g_desc(j, b).wait()
            nxt = j + 2

            @pl.when(nxt < nchunk)
            def _():
                @pl.when(j >= 1)
                def _():
                    s_desc(j - 1, nxt % 3).wait()
                g_desc(nxt, nxt % 3).start()

            s_desc(j, b).start()
            return 0

        lax.fori_loop(0, nchunk, step, 0)
        for j in (nchunk - 3, nchunk - 2, nchunk - 1):
            s_desc(j, j % 3).wait()

    return pl.kernel(
        body,
        out_type=jax.ShapeDtypeStruct((epw * NW, D), jnp.float32),
        mesh=_sc_mesh(),
        scratch_types=[
            pltpu.VMEM((nchunk, chunk), jnp.int32),
            pltpu.VMEM((3, chunk, D), jnp.float32),
            pltpu.SemaphoreType.DMA((3,)),
            pltpu.SemaphoreType.DMA((3,)),
        ],
    )


@functools.cache
def _sc_scatter(epw, chunk, nchunk):
    def body(msg_hbm, idx_hbm, out_hbm, idx_v, rows_v, hacc, lsem, asem):
        cid = lax.axis_index("c")
        sid = lax.axis_index("s")
        wid = sid * NC + cid

        def zrow(i, _):
            def zseg(k, _):
                rows_v[0, i, pl.ds(k * 16, 16)] = jnp.zeros(
                    (16,), jnp.float32)
                return 0
            lax.fori_loop(0, D // 16, zseg, 0)
            return 0

        lax.fori_loop(0, ZCH, zrow, 0)

        def zcopy(t, _):
            j = sid + t * NS

            @pl.when(j < NZCH)
            def _():
                pltpu.sync_copy(
                    rows_v.at[0].at[pl.ds(0, ZCH)],
                    hacc.at[pl.ds(j * ZCH, ZCH)])
            return 0

        lax.fori_loop(0, ZITER, zcopy, 0)
        plsc.subcore_barrier()

        pltpu.sync_copy(idx_hbm.at[wid], idx_v)
        base = wid * epw

        def l_desc(j, b):
            return pltpu.make_async_copy(
                msg_hbm.at[pl.ds(base + j * chunk, chunk)],
                rows_v.at[b].at[pl.ds(0, chunk)], lsem.at[b])

        def a_desc(j, b):
            return pltpu.make_async_copy(
                rows_v.at[b].at[pl.ds(0, chunk)], hacc.at[idx_v.at[j]],
                asem.at[b])

        l_desc(0, 0).start()
        l_desc(1, 1).start()

        def step(j, _):
            b = j % 3
            l_desc(j, b).wait()
            nxt = j + 2

            @pl.when(nxt < nchunk)
            def _():
                @pl.when(j >= 1)
                def _():
                    a_desc(j - 1, nxt % 3).wait()
                l_desc(nxt, nxt % 3).start()

            a_desc(j, b).start(add=True)
            return 0

        lax.fori_loop(0, nchunk, step, 0)
        for j in (nchunk - 3, nchunk - 2, nchunk - 1):
            a_desc(j, j % 3).wait()
        plsc.subcore_barrier()

        def ecopy(t, _):
            j = sid + t * NS

            @pl.when(j < NZCH)
            def _():
                sl = pl.ds(j * ZCH, ZCH)
                pltpu.sync_copy(hacc.at[sl], out_hbm.at[cid].at[sl])
            return 0

        lax.fori_loop(0, ZITER, ecopy, 0)

    buf_rows = max(chunk, ZCH)
    return pl.kernel(
        body,
        out_type=jax.ShapeDtypeStruct((NC, N, D), jnp.float32),
        mesh=_sc_mesh(),
        scratch_types=[
            pltpu.VMEM((nchunk, chunk), jnp.int32),
            pltpu.VMEM((3, buf_rows, D), jnp.float32),
            pltpu.VMEM_SHARED((N, D), jnp.float32),
            pltpu.SemaphoreType.DMA((3,)),
            pltpu.SemaphoreType.DMA((3,)),
        ],
    )


def _stats_body(xg_ref, a_ref, w1x_ref, w1a_ref, b1_ref, out_ref, e_ref):
    i = pl.program_id(0)
    a = a_ref[...]
    e_ref[...] = jnp.maximum(a, 0.0)
    xgb = xg_ref[...].astype(jnp.bfloat16)
    z = jnp.dot(xgb, w1x_ref[...], preferred_element_type=jnp.float32)
    z = z + jnp.dot(a.astype(jnp.bfloat16), w1a_ref[...],
                    preferred_element_type=jnp.float32)
    z = z + b1_ref[...]
    s1 = jnp.sum(z, axis=0, keepdims=True)
    s2 = jnp.sum(z * z, axis=0, keepdims=True)
    blk = jnp.concatenate(
        [s1, s2, jnp.zeros((6, EMB), jnp.float32)], axis=0)

    @pl.when(i == 0)
    def _():
        out_ref[...] = blk

    @pl.when(i > 0)
    def _():
        out_ref[...] = out_ref[...] + blk


def _tc_stats(xg, a, w1x, w1a, b1r):
    return pl.pallas_call(
        _stats_body,
        grid=(NEB,),
        in_specs=[
            pl.BlockSpec((EB, D), lambda i: (i, 0)),
            pl.BlockSpec((EB, DE), lambda i: (i, 0)),
            pl.BlockSpec((D, EMB), lambda i: (0, 0)),
            pl.BlockSpec((DE, EMB), lambda i: (0, 0)),
            pl.BlockSpec((1, EMB), lambda i: (0, 0)),
        ],
        out_specs=[
            pl.BlockSpec((8, EMB), lambda i: (0, 0)),
            pl.BlockSpec((EB, DE), lambda i: (i, 0)),
        ],
        out_shape=[
            jax.ShapeDtypeStruct((8, EMB), jnp.float32),
            jax.ShapeDtypeStruct((E, DE), jnp.float32),
        ],
        compiler_params=pltpu.CompilerParams(
            dimension_semantics=("arbitrary",)),
    )(xg, a, w1x, w1a, b1r)


def _main_body(st_ref, xg_ref, a_ref, w1x_ref, w1a_ref, b1_ref,
               g_ref, be_ref, w2_ref, b2_ref, msg_ref):
    st = st_ref[...]
    mu = st[0:1, :] * (1.0 / E)
    ex2 = st[1:2, :] * (1.0 / E)
    var = ex2 - mu * mu
    scale = lax.rsqrt(var + 1e-5) * g_ref[...]
    xgb = xg_ref[...].astype(jnp.bfloat16)
    z = jnp.dot(xgb, w1x_ref[...], preferred_element_type=jnp.float32)
    z = z + jnp.dot(a_ref[...].astype(jnp.bfloat16), w1a_ref[...],
                    preferred_element_type=jnp.float32)
    z = z + b1_ref[...]
    zn = (z - mu) * scale + be_ref[...]
    r = jnp.maximum(zn, 0.0).astype(jnp.bfloat16)
    msg_ref[...] = jnp.dot(r, w2_ref[...],
                           preferred_element_type=jnp.float32) + b2_ref[...]


def _tc_main(st, xg, a, w1x, w1a, b1r, gr, br, w2, b2r):
    return pl.pallas_call(
        _main_body,
        grid=(NEB,),
        in_specs=[
            pl.BlockSpec((8, EMB), lambda i: (0, 0)),
            pl.BlockSpec((EB, D), lambda i: (i, 0)),
            pl.BlockSpec((EB, DE), lambda i: (i, 0)),
            pl.BlockSpec((D, EMB), lambda i: (0, 0)),
            pl.BlockSpec((DE, EMB), lambda i: (0, 0)),
            pl.BlockSpec((1, EMB), lambda i: (0, 0)),
            pl.BlockSpec((1, EMB), lambda i: (0, 0)),
            pl.BlockSpec((1, EMB), lambda i: (0, 0)),
            pl.BlockSpec((EMB, D), lambda i: (0, 0)),
            pl.BlockSpec((1, D), lambda i: (0, 0)),
        ],
        out_specs=pl.BlockSpec((EB, D), lambda i: (i, 0)),
        out_shape=jax.ShapeDtypeStruct((E, D), jnp.float32),
        compiler_params=pltpu.CompilerParams(
            dimension_semantics=("arbitrary",)),
    )(st, xg, a, w1x, w1a, b1r, gr, br, w2, b2r)


def _final_body(p_ref, h_ref):
    h_ref[...] = jnp.maximum(p_ref[0] + p_ref[1], 0.0)


def _tc_final(partials):
    nb = 2000
    return pl.pallas_call(
        _final_body,
        grid=(N // nb,),
        in_specs=[pl.BlockSpec((NC, nb, D), lambda i: (0, i, 0))],
        out_specs=pl.BlockSpec((nb, D), lambda i: (i, 0)),
        out_shape=jax.ShapeDtypeStruct((N, D), jnp.float32),
    )(partials)


def kernel(x, edge_index, edge_attr, W1, b1, gamma, beta, W2, b2):
    src3d = edge_index[0].reshape(NW, NCHUNK, CHUNK)
    dst3d = edge_index[1].reshape(NW, NCHUNK, CHUNK)
    w1x = W1[:D]
    w1a = W1[D:]
    b1r = b1.reshape(1, EMB)
    gr = gamma.reshape(1, EMB)
    br = beta.reshape(1, EMB)
    b2r = b2.reshape(1, D)

    w1xb = w1x.astype(jnp.bfloat16)
    w1ab = w1a.astype(jnp.bfloat16)
    w2b = W2.astype(jnp.bfloat16)
    xg = _sc_gather(EPW, CHUNK, NCHUNK)(x, src3d)
    st, e_out = _tc_stats(xg, edge_attr, w1xb, w1ab, b1r)
    msg = _tc_main(st, xg, edge_attr, w1xb, w1ab, b1r, gr, br, w2b, b2r)
    partials = _sc_scatter(EPW, CHUNK, NCHUNK)(msg, dst3d)
    h = _tc_final(partials)
    return (h, e_out)


# speedup vs baseline: 1.0915x; 1.0028x over previous
import functools

import jax
import jax.numpy as jnp
from jax import lax
from jax.experimental import pallas as pl
from jax.experimental.pallas import tpu as pltpu
from jax.experimental.pallas import tpu_sc as plsc

N = 10000
E = 320000
D = 128
DE = 16
EMB = D + DE

NC = 2
NS = 16
NW = NC * NS
EPW = E // NW
CHUNK = 80
NCHUNK = EPW // CHUNK

EB = 2560
NEB = E // EB

ZCH = 80
NZCH = N // ZCH
ZITER = (NZCH + NS - 1) // NS


def _sc_mesh():
    return plsc.VectorSubcoreMesh(core_axis_name="c", subcore_axis_name="s")


@functools.cache
def _sc_gather(epw, chunk, nchunk):
    def body(x_hbm, idx_hbm, out_hbm, idx_v, rows_v, gsem, ssem):
        wid = lax.axis_index("s") * NC + lax.axis_index("c")
        pltpu.sync_copy(idx_hbm.at[wid], idx_v)
        base = wid * epw

        def g_desc(j, b):
            return pltpu.make_async_copy(
                x_hbm.at[idx_v.at[j]], rows_v.at[b], gsem.at[b])

        def s_desc(j, b):
            return pltpu.make_async_copy(
                rows_v.at[b], out_hbm.at[pl.ds(base + j * chunk, chunk)],
                ssem.at[b])

        nbuf = 5
        for k in range(nbuf - 1):
            g_desc(k, k).start()

        def step(j, _):
            b = j % nbuf
            g_desc(j, b).wait()
            nxt = j + nbuf - 1

            @pl.when(nxt < nchunk)
            def _():
                @pl.when(j >= 1)
                def _():
                    s_desc(j - 1, nxt % nbuf).wait()
                g_desc(nxt, nxt % nbuf).start()

            s_desc(j, b).start()
            return 0

        lax.fori_loop(0, nchunk, step, 0)
        for j in range(nchunk - nbuf, nchunk):
            s_desc(j, j % nbuf).wait()

    return pl.kernel(
        body,
        out_type=jax.ShapeDtypeStruct((epw * NW, D), jnp.float32),
        mesh=_sc_mesh(),
        scratch_types=[
            pltpu.VMEM((nchunk, chunk), jnp.int32),
            pltpu.VMEM((5, chunk, D), jnp.float32),
            pltpu.SemaphoreType.DMA((5,)),
            pltpu.SemaphoreType.DMA((5,)),
        ],
    )


@functools.cache
def _sc_scatter(epw, chunk, nchunk):
    def body(msg_hbm, idx_hbm, out_hbm, idx_v, rows_v, hacc, lsem, asem):
        cid = lax.axis_index("c")
        sid = lax.axis_index("s")
        wid = sid * NC + cid

        def zrow(i, _):
            def zseg(k, _):
                rows_v[0, i, pl.ds(k * 16, 16)] = jnp.zeros(
                    (16,), jnp.float32)
                return 0
            lax.fori_loop(0, D // 16, zseg, 0)
            return 0

        lax.fori_loop(0, ZCH, zrow, 0)

        def zcopy(t, _):
            j = sid + t * NS

            @pl.when(j < NZCH)
            def _():
                pltpu.sync_copy(
                    rows_v.at[0].at[pl.ds(0, ZCH)],
                    hacc.at[pl.ds(j * ZCH, ZCH)])
            return 0

        lax.fori_loop(0, ZITER, zcopy, 0)
        plsc.subcore_barrier()

        pltpu.sync_copy(idx_hbm.at[wid], idx_v)
        base = wid * epw

        def l_desc(j, b):
            return pltpu.make_async_copy(
                msg_hbm.at[pl.ds(base + j * chunk, chunk)],
                rows_v.at[b].at[pl.ds(0, chunk)], lsem.at[b])

        def a_desc(j, b):
            return pltpu.make_async_copy(
                rows_v.at[b].at[pl.ds(0, chunk)], hacc.at[idx_v.at[j]],
                asem.at[b])

        l_desc(0, 0).start()
        l_desc(1, 1).start()

        def step(j, _):
            b = j % 3
            l_desc(j, b).wait()
            nxt = j + 2

            @pl.when(nxt < nchunk)
            def _():
                @pl.when(j >= 1)
                def _():
                    a_desc(j - 1, nxt % 3).wait()
                l_desc(nxt, nxt % 3).start()

            a_desc(j, b).start(add=True)
            return 0

        lax.fori_loop(0, nchunk, step, 0)
        for j in (nchunk - 3, nchunk - 2, nchunk - 1):
            a_desc(j, j % 3).wait()
        plsc.subcore_barrier()

        def ecopy(t, _):
            j = sid + t * NS

            @pl.when(j < NZCH)
            def _():
                sl = pl.ds(j * ZCH, ZCH)
                pltpu.sync_copy(hacc.at[sl], out_hbm.at[cid].at[sl])
            return 0

        lax.fori_loop(0, ZITER, ecopy, 0)

    buf_rows = max(chunk, ZCH)
    return pl.kernel(
        body,
        out_type=jax.ShapeDtypeStruct((NC, N, D), jnp.float32),
        mesh=_sc_mesh(),
        scratch_types=[
            pltpu.VMEM((nchunk, chunk), jnp.int32),
            pltpu.VMEM((3, buf_rows, D), jnp.float32),
            pltpu.VMEM_SHARED((N, D), jnp.float32),
            pltpu.SemaphoreType.DMA((3,)),
            pltpu.SemaphoreType.DMA((3,)),
        ],
    )


def _stats_body(xg_ref, a_ref, w1x_ref, w1a_ref, b1_ref, out_ref, e_ref):
    i = pl.program_id(0)
    a = a_ref[...]
    e_ref[...] = jnp.maximum(a, 0.0)
    xgb = xg_ref[...].astype(jnp.bfloat16)
    z = jnp.dot(xgb, w1x_ref[...], preferred_element_type=jnp.float32)
    z = z + jnp.dot(a.astype(jnp.bfloat16), w1a_ref[...],
                    preferred_element_type=jnp.float32)
    z = z + b1_ref[...]
    s1 = jnp.sum(z, axis=0, keepdims=True)
    s2 = jnp.sum(z * z, axis=0, keepdims=True)
    blk = jnp.concatenate(
        [s1, s2, jnp.zeros((6, EMB), jnp.float32)], axis=0)

    @pl.when(i == 0)
    def _():
        out_ref[...] = blk

    @pl.when(i > 0)
    def _():
        out_ref[...] = out_ref[...] + blk


def _tc_stats(xg, a, w1x, w1a, b1r):
    return pl.pallas_call(
        _stats_body,
        grid=(NEB,),
        in_specs=[
            pl.BlockSpec((EB, D), lambda i: (i, 0)),
            pl.BlockSpec((EB, DE), lambda i: (i, 0)),
            pl.BlockSpec((D, EMB), lambda i: (0, 0)),
            pl.BlockSpec((DE, EMB), lambda i: (0, 0)),
            pl.BlockSpec((1, EMB), lambda i: (0, 0)),
        ],
        out_specs=[
            pl.BlockSpec((8, EMB), lambda i: (0, 0)),
            pl.BlockSpec((EB, DE), lambda i: (i, 0)),
        ],
        out_shape=[
            jax.ShapeDtypeStruct((8, EMB), jnp.float32),
            jax.ShapeDtypeStruct((E, DE), jnp.float32),
        ],
        compiler_params=pltpu.CompilerParams(
            dimension_semantics=("arbitrary",)),
    )(xg, a, w1x, w1a, b1r)


def _main_body(st_ref, xg_ref, a_ref, w1x_ref, w1a_ref, b1_ref,
               g_ref, be_ref, w2_ref, b2_ref, msg_ref):
    st = st_ref[...]
    mu = st[0:1, :] * (1.0 / E)
    ex2 = st[1:2, :] * (1.0 / E)
    var = ex2 - mu * mu
    scale = lax.rsqrt(var + 1e-5) * g_ref[...]
    xgb = xg_ref[...].astype(jnp.bfloat16)
    z = jnp.dot(xgb, w1x_ref[...], preferred_element_type=jnp.float32)
    z = z + jnp.dot(a_ref[...].astype(jnp.bfloat16), w1a_ref[...],
                    preferred_element_type=jnp.float32)
    z = z + b1_ref[...]
    zn = (z - mu) * scale + be_ref[...]
    r = jnp.maximum(zn, 0.0).astype(jnp.bfloat16)
    msg_ref[...] = jnp.dot(r, w2_ref[...],
                           preferred_element_type=jnp.float32) + b2_ref[...]


def _tc_main(st, xg, a, w1x, w1a, b1r, gr, br, w2, b2r):
    return pl.pallas_call(
        _main_body,
        grid=(NEB,),
        in_specs=[
            pl.BlockSpec((8, EMB), lambda i: (0, 0)),
            pl.BlockSpec((EB, D), lambda i: (i, 0)),
            pl.BlockSpec((EB, DE), lambda i: (i, 0)),
            pl.BlockSpec((D, EMB), lambda i: (0, 0)),
            pl.BlockSpec((DE, EMB), lambda i: (0, 0)),
            pl.BlockSpec((1, EMB), lambda i: (0, 0)),
            pl.BlockSpec((1, EMB), lambda i: (0, 0)),
            pl.BlockSpec((1, EMB), lambda i: (0, 0)),
            pl.BlockSpec((EMB, D), lambda i: (0, 0)),
            pl.BlockSpec((1, D), lambda i: (0, 0)),
        ],
        out_specs=pl.BlockSpec((EB, D), lambda i: (i, 0)),
        out_shape=jax.ShapeDtypeStruct((E, D), jnp.float32),
        compiler_params=pltpu.CompilerParams(
            dimension_semantics=("arbitrary",)),
    )(st, xg, a, w1x, w1a, b1r, gr, br, w2, b2r)


def _final_body(p_ref, h_ref):
    h_ref[...] = jnp.maximum(p_ref[0] + p_ref[1], 0.0)


def _tc_final(partials):
    nb = 2000
    return pl.pallas_call(
        _final_body,
        grid=(N // nb,),
        in_specs=[pl.BlockSpec((NC, nb, D), lambda i: (0, i, 0))],
        out_specs=pl.BlockSpec((nb, D), lambda i: (i, 0)),
        out_shape=jax.ShapeDtypeStruct((N, D), jnp.float32),
    )(partials)


def kernel(x, edge_index, edge_attr, W1, b1, gamma, beta, W2, b2):
    src3d = edge_index[0].reshape(NW, NCHUNK, CHUNK)
    dst3d = edge_index[1].reshape(NW, NCHUNK, CHUNK)
    w1x = W1[:D]
    w1a = W1[D:]
    b1r = b1.reshape(1, EMB)
    gr = gamma.reshape(1, EMB)
    br = beta.reshape(1, EMB)
    b2r = b2.reshape(1, D)

    w1xb = w1x.astype(jnp.bfloat16)
    w1ab = w1a.astype(jnp.bfloat16)
    w2b = W2.astype(jnp.bfloat16)
    xg = _sc_gather(EPW, CHUNK, NCHUNK)(x, src3d)
    st, e_out = _tc_stats(xg, edge_attr, w1xb, w1ab, b1r)
    msg = _tc_main(st, xg, edge_attr, w1xb, w1ab, b1r, gr, br, w2b, b2r)
    partials = _sc_scatter(EPW, CHUNK, NCHUNK)(msg, dst3d)
    h = _tc_final(partials)
    return (h, e_out)


# speedup vs baseline: 1.1815x; 1.0824x over previous
import functools

import jax
import jax.numpy as jnp
from jax import lax
from jax.experimental import pallas as pl
from jax.experimental.pallas import tpu as pltpu
from jax.experimental.pallas import tpu_sc as plsc

N = 10000
E = 320000
D = 128
DE = 16
EMB = D + DE

NC = 2
NS = 16
NW = NC * NS
EPW = E // NW
CHUNK = 80
NCHUNK = EPW // CHUNK

EB = 4000
NEB = E // EB

ZCH = 80
NZCH = N // ZCH
ZITER = (NZCH + NS - 1) // NS


def _sc_mesh():
    return plsc.VectorSubcoreMesh(core_axis_name="c", subcore_axis_name="s")


@functools.cache
def _sc_gather(epw, chunk, nchunk):
    def body(x_hbm, idx_hbm, out_hbm, idx_v, rows_v, gsem, ssem):
        wid = lax.axis_index("s") * NC + lax.axis_index("c")
        pltpu.sync_copy(idx_hbm.at[wid], idx_v)
        base = wid * epw

        def g_desc(j, b):
            return pltpu.make_async_copy(
                x_hbm.at[idx_v.at[j]], rows_v.at[b], gsem.at[b])

        def s_desc(j, b):
            return pltpu.make_async_copy(
                rows_v.at[b], out_hbm.at[pl.ds(base + j * chunk, chunk)],
                ssem.at[b])

        nbuf = 5
        for k in range(nbuf - 1):
            g_desc(k, k).start()

        def step(j, _):
            b = j % nbuf
            g_desc(j, b).wait()
            nxt = j + nbuf - 1

            @pl.when(nxt < nchunk)
            def _():
                @pl.when(j >= 1)
                def _():
                    s_desc(j - 1, nxt % nbuf).wait()
                g_desc(nxt, nxt % nbuf).start()

            s_desc(j, b).start()
            return 0

        lax.fori_loop(0, nchunk, step, 0)
        for j in range(nchunk - nbuf, nchunk):
            s_desc(j, j % nbuf).wait()

    return pl.kernel(
        body,
        out_type=jax.ShapeDtypeStruct((epw * NW, D), jnp.float32),
        mesh=_sc_mesh(),
        scratch_types=[
            pltpu.VMEM((nchunk, chunk), jnp.int32),
            pltpu.VMEM((5, chunk, D), jnp.float32),
            pltpu.SemaphoreType.DMA((5,)),
            pltpu.SemaphoreType.DMA((5,)),
        ],
    )


@functools.cache
def _sc_scatter(epw, chunk, nchunk):
    def body(msg_hbm, idx_hbm, out_hbm, idx_v, rows_v, hacc, lsem, asem):
        cid = lax.axis_index("c")
        sid = lax.axis_index("s")
        wid = sid * NC + cid

        def zrow(i, _):
            def zseg(k, _):
                rows_v[0, i, pl.ds(k * 16, 16)] = jnp.zeros(
                    (16,), jnp.float32)
                return 0
            lax.fori_loop(0, D // 16, zseg, 0)
            return 0

        lax.fori_loop(0, ZCH, zrow, 0)

        def zcopy(t, _):
            j = sid + t * NS

            @pl.when(j < NZCH)
            def _():
                pltpu.sync_copy(
                    rows_v.at[0].at[pl.ds(0, ZCH)],
                    hacc.at[pl.ds(j * ZCH, ZCH)])
            return 0

        lax.fori_loop(0, ZITER, zcopy, 0)
        plsc.subcore_barrier()

        pltpu.sync_copy(idx_hbm.at[wid], idx_v)
        base = wid * epw

        def l_desc(j, b):
            return pltpu.make_async_copy(
                msg_hbm.at[pl.ds(base + j * chunk, chunk)],
                rows_v.at[b].at[pl.ds(0, chunk)], lsem.at[b])

        def a_desc(j, b):
            return pltpu.make_async_copy(
                rows_v.at[b].at[pl.ds(0, chunk)], hacc.at[idx_v.at[j]],
                asem.at[b])

        l_desc(0, 0).start()
        l_desc(1, 1).start()

        def step(j, _):
            b = j % 3
            l_desc(j, b).wait()
            nxt = j + 2

            @pl.when(nxt < nchunk)
            def _():
                @pl.when(j >= 1)
                def _():
                    a_desc(j - 1, nxt % 3).wait()
                l_desc(nxt, nxt % 3).start()

            a_desc(j, b).start(add=True)
            return 0

        lax.fori_loop(0, nchunk, step, 0)
        for j in (nchunk - 3, nchunk - 2, nchunk - 1):
            a_desc(j, j % 3).wait()
        plsc.subcore_barrier()

        def ecopy(t, _):
            j = sid + t * NS

            @pl.when(j < NZCH)
            def _():
                sl = pl.ds(j * ZCH, ZCH)
                pltpu.sync_copy(hacc.at[sl], out_hbm.at[cid].at[sl])
            return 0

        lax.fori_loop(0, ZITER, ecopy, 0)

    buf_rows = max(chunk, ZCH)
    return pl.kernel(
        body,
        out_type=jax.ShapeDtypeStruct((NC, N, D), jnp.float32),
        mesh=_sc_mesh(),
        scratch_types=[
            pltpu.VMEM((nchunk, chunk), jnp.int32),
            pltpu.VMEM((3, buf_rows, D), jnp.float32),
            pltpu.VMEM_SHARED((N, D), jnp.float32),
            pltpu.SemaphoreType.DMA((3,)),
            pltpu.SemaphoreType.DMA((3,)),
        ],
    )


def _stats_body(xg_ref, a_ref, w1x_ref, w1a_ref, b1_ref, out_ref, e_ref):
    i = pl.program_id(0)
    a = a_ref[...]
    e_ref[...] = jnp.maximum(a, 0.0)
    xgb = xg_ref[...].astype(jnp.bfloat16)
    z = jnp.dot(xgb, w1x_ref[...], preferred_element_type=jnp.float32)
    z = z + jnp.dot(a.astype(jnp.bfloat16), w1a_ref[...],
                    preferred_element_type=jnp.float32)
    z = z + b1_ref[...]
    s1 = jnp.sum(z, axis=0, keepdims=True)
    s2 = jnp.sum(z * z, axis=0, keepdims=True)
    blk = jnp.concatenate(
        [s1, s2, jnp.zeros((6, EMB), jnp.float32)], axis=0)

    @pl.when(i == 0)
    def _():
        out_ref[...] = blk

    @pl.when(i > 0)
    def _():
        out_ref[...] = out_ref[...] + blk


def _tc_stats(xg, a, w1x, w1a, b1r):
    return pl.pallas_call(
        _stats_body,
        grid=(NEB,),
        in_specs=[
            pl.BlockSpec((EB, D), lambda i: (i, 0)),
            pl.BlockSpec((EB, DE), lambda i: (i, 0)),
            pl.BlockSpec((D, EMB), lambda i: (0, 0)),
            pl.BlockSpec((DE, EMB), lambda i: (0, 0)),
            pl.BlockSpec((1, EMB), lambda i: (0, 0)),
        ],
        out_specs=[
            pl.BlockSpec((8, EMB), lambda i: (0, 0)),
            pl.BlockSpec((EB, DE), lambda i: (i, 0)),
        ],
        out_shape=[
            jax.ShapeDtypeStruct((8, EMB), jnp.float32),
            jax.ShapeDtypeStruct((E, DE), jnp.float32),
        ],
        compiler_params=pltpu.CompilerParams(
            dimension_semantics=("arbitrary",)),
    )(xg, a, w1x, w1a, b1r)


def _main_body(st_ref, xg_ref, a_ref, w1x_ref, w1a_ref, b1_ref,
               g_ref, be_ref, w2_ref, b2_ref, msg_ref):
    st = st_ref[...]
    mu = st[0:1, :] * (1.0 / E)
    ex2 = st[1:2, :] * (1.0 / E)
    var = ex2 - mu * mu
    scale = lax.rsqrt(var + 1e-5) * g_ref[...]
    xgb = xg_ref[...].astype(jnp.bfloat16)
    z = jnp.dot(xgb, w1x_ref[...], preferred_element_type=jnp.float32)
    z = z + jnp.dot(a_ref[...].astype(jnp.bfloat16), w1a_ref[...],
                    preferred_element_type=jnp.float32)
    z = z + b1_ref[...]
    zn = (z - mu) * scale + be_ref[...]
    r = jnp.maximum(zn, 0.0).astype(jnp.bfloat16)
    msg_ref[...] = jnp.dot(r, w2_ref[...],
                           preferred_element_type=jnp.float32) + b2_ref[...]


def _tc_main(st, xg, a, w1x, w1a, b1r, gr, br, w2, b2r):
    return pl.pallas_call(
        _main_body,
        grid=(NEB,),
        in_specs=[
            pl.BlockSpec((8, EMB), lambda i: (0, 0)),
            pl.BlockSpec((EB, D), lambda i: (i, 0)),
            pl.BlockSpec((EB, DE), lambda i: (i, 0)),
            pl.BlockSpec((D, EMB), lambda i: (0, 0)),
            pl.BlockSpec((DE, EMB), lambda i: (0, 0)),
            pl.BlockSpec((1, EMB), lambda i: (0, 0)),
            pl.BlockSpec((1, EMB), lambda i: (0, 0)),
            pl.BlockSpec((1, EMB), lambda i: (0, 0)),
            pl.BlockSpec((EMB, D), lambda i: (0, 0)),
            pl.BlockSpec((1, D), lambda i: (0, 0)),
        ],
        out_specs=pl.BlockSpec((EB, D), lambda i: (i, 0)),
        out_shape=jax.ShapeDtypeStruct((E, D), jnp.float32),
        compiler_params=pltpu.CompilerParams(
            dimension_semantics=("arbitrary",)),
    )(st, xg, a, w1x, w1a, b1r, gr, br, w2, b2r)


def _final_body(p_ref, h_ref):
    h_ref[...] = jnp.maximum(p_ref[0] + p_ref[1], 0.0)


def _tc_final(partials):
    nb = 2000
    return pl.pallas_call(
        _final_body,
        grid=(N // nb,),
        in_specs=[pl.BlockSpec((NC, nb, D), lambda i: (0, i, 0))],
        out_specs=pl.BlockSpec((nb, D), lambda i: (i, 0)),
        out_shape=jax.ShapeDtypeStruct((N, D), jnp.float32),
    )(partials)


def kernel(x, edge_index, edge_attr, W1, b1, gamma, beta, W2, b2):
    src3d = edge_index[0].reshape(NW, NCHUNK, CHUNK)
    dst3d = edge_index[1].reshape(NW, NCHUNK, CHUNK)
    w1x = W1[:D]
    w1a = W1[D:]
    b1r = b1.reshape(1, EMB)
    gr = gamma.reshape(1, EMB)
    br = beta.reshape(1, EMB)
    b2r = b2.reshape(1, D)

    w1xb = w1x.astype(jnp.bfloat16)
    w1ab = w1a.astype(jnp.bfloat16)
    w2b = W2.astype(jnp.bfloat16)
    xg = _sc_gather(EPW, CHUNK, NCHUNK)(x, src3d)
    st, e_out = _tc_stats(xg, edge_attr, w1xb, w1ab, b1r)
    msg = _tc_main(st, xg, edge_attr, w1xb, w1ab, b1r, gr, br, w2b, b2r)
    partials = _sc_scatter(EPW, CHUNK, NCHUNK)(msg, dst3d)
    h = _tc_final(partials)
    return (h, e_out)


# speedup vs baseline: 1.2747x; 1.0789x over previous
import functools

import jax
import jax.numpy as jnp
from jax import lax
from jax.experimental import pallas as pl
from jax.experimental.pallas import tpu as pltpu
from jax.experimental.pallas import tpu_sc as plsc

N = 10000
E = 320000
D = 128
DE = 16
EMB = D + DE

NC = 2
NS = 16
NW = NC * NS
EPW = E // NW
CHUNK = 80
NCHUNK = EPW // CHUNK

EB = 8000
NEB = E // EB

ZCH = 80
NZCH = N // ZCH
ZITER = (NZCH + NS - 1) // NS


def _sc_mesh():
    return plsc.VectorSubcoreMesh(core_axis_name="c", subcore_axis_name="s")


@functools.cache
def _sc_gather(epw, chunk, nchunk):
    def body(x_hbm, idx_hbm, out_hbm, idx_v, rows_v, gsem, ssem):
        wid = lax.axis_index("s") * NC + lax.axis_index("c")
        pltpu.sync_copy(idx_hbm.at[wid], idx_v)
        base = wid * epw

        def g_desc(j, b):
            return pltpu.make_async_copy(
                x_hbm.at[idx_v.at[j]], rows_v.at[b], gsem.at[b])

        def s_desc(j, b):
            return pltpu.make_async_copy(
                rows_v.at[b], out_hbm.at[pl.ds(base + j * chunk, chunk)],
                ssem.at[b])

        nbuf = 5
        for k in range(nbuf - 1):
            g_desc(k, k).start()

        def step(j, _):
            b = j % nbuf
            g_desc(j, b).wait()
            nxt = j + nbuf - 1

            @pl.when(nxt < nchunk)
            def _():
                @pl.when(j >= 1)
                def _():
                    s_desc(j - 1, nxt % nbuf).wait()
                g_desc(nxt, nxt % nbuf).start()

            s_desc(j, b).start()
            return 0

        lax.fori_loop(0, nchunk, step, 0)
        for j in range(nchunk - nbuf, nchunk):
            s_desc(j, j % nbuf).wait()

    return pl.kernel(
        body,
        out_type=jax.ShapeDtypeStruct((epw * NW, D), jnp.float32),
        mesh=_sc_mesh(),
        scratch_types=[
            pltpu.VMEM((nchunk, chunk), jnp.int32),
            pltpu.VMEM((5, chunk, D), jnp.float32),
            pltpu.SemaphoreType.DMA((5,)),
            pltpu.SemaphoreType.DMA((5,)),
        ],
    )


@functools.cache
def _sc_scatter(epw, chunk, nchunk):
    def body(msg_hbm, idx_hbm, out_hbm, idx_v, rows_v, hacc, lsem, asem):
        cid = lax.axis_index("c")
        sid = lax.axis_index("s")
        wid = sid * NC + cid

        def zrow(i, _):
            def zseg(k, _):
                rows_v[0, i, pl.ds(k * 16, 16)] = jnp.zeros(
                    (16,), jnp.float32)
                return 0
            lax.fori_loop(0, D // 16, zseg, 0)
            return 0

        lax.fori_loop(0, ZCH, zrow, 0)

        def zcopy(t, _):
            j = sid + t * NS

            @pl.when(j < NZCH)
            def _():
                pltpu.sync_copy(
                    rows_v.at[0].at[pl.ds(0, ZCH)],
                    hacc.at[pl.ds(j * ZCH, ZCH)])
            return 0

        lax.fori_loop(0, ZITER, zcopy, 0)
        plsc.subcore_barrier()

        pltpu.sync_copy(idx_hbm.at[wid], idx_v)
        base = wid * epw

        def l_desc(j, b):
            return pltpu.make_async_copy(
                msg_hbm.at[pl.ds(base + j * chunk, chunk)],
                rows_v.at[b].at[pl.ds(0, chunk)], lsem.at[b])

        def a_desc(j, b):
            return pltpu.make_async_copy(
                rows_v.at[b].at[pl.ds(0, chunk)], hacc.at[idx_v.at[j]],
                asem.at[b])

        l_desc(0, 0).start()
        l_desc(1, 1).start()

        def step(j, _):
            b = j % 3
            l_desc(j, b).wait()
            nxt = j + 2

            @pl.when(nxt < nchunk)
            def _():
                @pl.when(j >= 1)
                def _():
                    a_desc(j - 1, nxt % 3).wait()
                l_desc(nxt, nxt % 3).start()

            a_desc(j, b).start(add=True)
            return 0

        lax.fori_loop(0, nchunk, step, 0)
        for j in (nchunk - 3, nchunk - 2, nchunk - 1):
            a_desc(j, j % 3).wait()
        plsc.subcore_barrier()

        def ecopy(t, _):
            j = sid + t * NS

            @pl.when(j < NZCH)
            def _():
                sl = pl.ds(j * ZCH, ZCH)
                pltpu.sync_copy(hacc.at[sl], out_hbm.at[cid].at[sl])
            return 0

        lax.fori_loop(0, ZITER, ecopy, 0)

    buf_rows = max(chunk, ZCH)
    return pl.kernel(
        body,
        out_type=jax.ShapeDtypeStruct((NC, N, D), jnp.float32),
        mesh=_sc_mesh(),
        scratch_types=[
            pltpu.VMEM((nchunk, chunk), jnp.int32),
            pltpu.VMEM((3, buf_rows, D), jnp.float32),
            pltpu.VMEM_SHARED((N, D), jnp.float32),
            pltpu.SemaphoreType.DMA((3,)),
            pltpu.SemaphoreType.DMA((3,)),
        ],
    )


def _stats_body(xg_ref, a_ref, w1x_ref, w1a_ref, b1_ref, out_ref, e_ref):
    i = pl.program_id(0)
    a = a_ref[...]
    e_ref[...] = jnp.maximum(a, 0.0)
    xgb = xg_ref[...].astype(jnp.bfloat16)
    z = jnp.dot(xgb, w1x_ref[...], preferred_element_type=jnp.float32)
    z = z + jnp.dot(a.astype(jnp.bfloat16), w1a_ref[...],
                    preferred_element_type=jnp.float32)
    z = z + b1_ref[...]
    s1 = jnp.sum(z, axis=0, keepdims=True)
    s2 = jnp.sum(z * z, axis=0, keepdims=True)
    blk = jnp.concatenate(
        [s1, s2, jnp.zeros((6, EMB), jnp.float32)], axis=0)

    @pl.when(i == 0)
    def _():
        out_ref[...] = blk

    @pl.when(i > 0)
    def _():
        out_ref[...] = out_ref[...] + blk


def _tc_stats(xg, a, w1x, w1a, b1r):
    return pl.pallas_call(
        _stats_body,
        grid=(NEB,),
        in_specs=[
            pl.BlockSpec((EB, D), lambda i: (i, 0)),
            pl.BlockSpec((EB, DE), lambda i: (i, 0)),
            pl.BlockSpec((D, EMB), lambda i: (0, 0)),
            pl.BlockSpec((DE, EMB), lambda i: (0, 0)),
            pl.BlockSpec((1, EMB), lambda i: (0, 0)),
        ],
        out_specs=[
            pl.BlockSpec((8, EMB), lambda i: (0, 0)),
            pl.BlockSpec((EB, DE), lambda i: (i, 0)),
        ],
        out_shape=[
            jax.ShapeDtypeStruct((8, EMB), jnp.float32),
            jax.ShapeDtypeStruct((E, DE), jnp.float32),
        ],
        compiler_params=pltpu.CompilerParams(
            dimension_semantics=("arbitrary",)),
    )(xg, a, w1x, w1a, b1r)


def _main_body(st_ref, xg_ref, a_ref, w1x_ref, w1a_ref, b1_ref,
               g_ref, be_ref, w2_ref, b2_ref, msg_ref):
    st = st_ref[...]
    mu = st[0:1, :] * (1.0 / E)
    ex2 = st[1:2, :] * (1.0 / E)
    var = ex2 - mu * mu
    scale = lax.rsqrt(var + 1e-5) * g_ref[...]
    xgb = xg_ref[...].astype(jnp.bfloat16)
    z = jnp.dot(xgb, w1x_ref[...], preferred_element_type=jnp.float32)
    z = z + jnp.dot(a_ref[...].astype(jnp.bfloat16), w1a_ref[...],
                    preferred_element_type=jnp.float32)
    z = z + b1_ref[...]
    zn = (z - mu) * scale + be_ref[...]
    r = jnp.maximum(zn, 0.0).astype(jnp.bfloat16)
    msg_ref[...] = jnp.dot(r, w2_ref[...],
                           preferred_element_type=jnp.float32) + b2_ref[...]


def _tc_main(st, xg, a, w1x, w1a, b1r, gr, br, w2, b2r):
    return pl.pallas_call(
        _main_body,
        grid=(NEB,),
        in_specs=[
            pl.BlockSpec((8, EMB), lambda i: (0, 0)),
            pl.BlockSpec((EB, D), lambda i: (i, 0)),
            pl.BlockSpec((EB, DE), lambda i: (i, 0)),
            pl.BlockSpec((D, EMB), lambda i: (0, 0)),
            pl.BlockSpec((DE, EMB), lambda i: (0, 0)),
            pl.BlockSpec((1, EMB), lambda i: (0, 0)),
            pl.BlockSpec((1, EMB), lambda i: (0, 0)),
            pl.BlockSpec((1, EMB), lambda i: (0, 0)),
            pl.BlockSpec((EMB, D), lambda i: (0, 0)),
            pl.BlockSpec((1, D), lambda i: (0, 0)),
        ],
        out_specs=pl.BlockSpec((EB, D), lambda i: (i, 0)),
        out_shape=jax.ShapeDtypeStruct((E, D), jnp.float32),
        compiler_params=pltpu.CompilerParams(
            dimension_semantics=("arbitrary",)),
    )(st, xg, a, w1x, w1a, b1r, gr, br, w2, b2r)


def _final_body(p_ref, h_ref):
    h_ref[...] = jnp.maximum(p_ref[0] + p_ref[1], 0.0)


def _tc_final(partials):
    nb = 2000
    return pl.pallas_call(
        _final_body,
        grid=(N // nb,),
        in_specs=[pl.BlockSpec((NC, nb, D), lambda i: (0, i, 0))],
        out_specs=pl.BlockSpec((nb, D), lambda i: (i, 0)),
        out_shape=jax.ShapeDtypeStruct((N, D), jnp.float32),
    )(partials)


def kernel(x, edge_index, edge_attr, W1, b1, gamma, beta, W2, b2):
    src3d = edge_index[0].reshape(NW, NCHUNK, CHUNK)
    dst3d = edge_index[1].reshape(NW, NCHUNK, CHUNK)
    w1x = W1[:D]
    w1a = W1[D:]
    b1r = b1.reshape(1, EMB)
    gr = gamma.reshape(1, EMB)
    br = beta.reshape(1, EMB)
    b2r = b2.reshape(1, D)

    w1xb = w1x.astype(jnp.bfloat16)
    w1ab = w1a.astype(jnp.bfloat16)
    w2b = W2.astype(jnp.bfloat16)
    xg = _sc_gather(EPW, CHUNK, NCHUNK)(x, src3d)
    st, e_out = _tc_stats(xg, edge_attr, w1xb, w1ab, b1r)
    msg = _tc_main(st, xg, edge_attr, w1xb, w1ab, b1r, gr, br, w2b, b2r)
    partials = _sc_scatter(EPW, CHUNK, NCHUNK)(msg, dst3d)
    h = _tc_final(partials)
    return (h, e_out)


# speedup vs baseline: 1.2938x; 1.0150x over previous
import functools

import jax
import jax.numpy as jnp
from jax import lax
from jax.experimental import pallas as pl
from jax.experimental.pallas import tpu as pltpu
from jax.experimental.pallas import tpu_sc as plsc

N = 10000
E = 320000
D = 128
DE = 16
EMB = D + DE

NC = 2
NS = 16
NW = NC * NS
EPW = E // NW
CHUNK = 80
NCHUNK = EPW // CHUNK

EB = 10000
NEB = E // EB

ZCH = 80
NZCH = N // ZCH
ZITER = (NZCH + NS - 1) // NS


def _sc_mesh():
    return plsc.VectorSubcoreMesh(core_axis_name="c", subcore_axis_name="s")


@functools.cache
def _sc_gather(epw, chunk, nchunk):
    def body(x_hbm, idx_hbm, out_hbm, idx_v, rows_v, gsem, ssem):
        wid = lax.axis_index("s") * NC + lax.axis_index("c")
        pltpu.sync_copy(idx_hbm.at[wid], idx_v)
        base = wid * epw

        def g_desc(j, b):
            return pltpu.make_async_copy(
                x_hbm.at[idx_v.at[j]], rows_v.at[b], gsem.at[b])

        def s_desc(j, b):
            return pltpu.make_async_copy(
                rows_v.at[b], out_hbm.at[pl.ds(base + j * chunk, chunk)],
                ssem.at[b])

        nbuf = 5
        for k in range(nbuf - 1):
            g_desc(k, k).start()

        def step(j, _):
            b = j % nbuf
            g_desc(j, b).wait()
            nxt = j + nbuf - 1

            @pl.when(nxt < nchunk)
            def _():
                @pl.when(j >= 1)
                def _():
                    s_desc(j - 1, nxt % nbuf).wait()
                g_desc(nxt, nxt % nbuf).start()

            s_desc(j, b).start()
            return 0

        lax.fori_loop(0, nchunk, step, 0)
        for j in range(nchunk - nbuf, nchunk):
            s_desc(j, j % nbuf).wait()

    return pl.kernel(
        body,
        out_type=jax.ShapeDtypeStruct((epw * NW, D), jnp.float32),
        mesh=_sc_mesh(),
        scratch_types=[
            pltpu.VMEM((nchunk, chunk), jnp.int32),
            pltpu.VMEM((5, chunk, D), jnp.float32),
            pltpu.SemaphoreType.DMA((5,)),
            pltpu.SemaphoreType.DMA((5,)),
        ],
    )


@functools.cache
def _sc_scatter(epw, chunk, nchunk):
    def body(msg_hbm, idx_hbm, out_hbm, idx_v, rows_v, hacc, lsem, asem):
        cid = lax.axis_index("c")
        sid = lax.axis_index("s")
        wid = sid * NC + cid

        def zrow(i, _):
            def zseg(k, _):
                rows_v[0, i, pl.ds(k * 16, 16)] = jnp.zeros(
                    (16,), jnp.float32)
                return 0
            lax.fori_loop(0, D // 16, zseg, 0)
            return 0

        lax.fori_loop(0, ZCH, zrow, 0)

        def zcopy(t, _):
            j = sid + t * NS

            @pl.when(j < NZCH)
            def _():
                pltpu.sync_copy(
                    rows_v.at[0].at[pl.ds(0, ZCH)],
                    hacc.at[pl.ds(j * ZCH, ZCH)])
            return 0

        lax.fori_loop(0, ZITER, zcopy, 0)
        plsc.subcore_barrier()

        pltpu.sync_copy(idx_hbm.at[wid], idx_v)
        base = wid * epw

        def l_desc(j, b):
            return pltpu.make_async_copy(
                msg_hbm.at[pl.ds(base + j * chunk, chunk)],
                rows_v.at[b].at[pl.ds(0, chunk)], lsem.at[b])

        def a_desc(j, b):
            return pltpu.make_async_copy(
                rows_v.at[b].at[pl.ds(0, chunk)], hacc.at[idx_v.at[j]],
                asem.at[b])

        l_desc(0, 0).start()
        l_desc(1, 1).start()

        def step(j, _):
            b = j % 3
            l_desc(j, b).wait()
            nxt = j + 2

            @pl.when(nxt < nchunk)
            def _():
                @pl.when(j >= 1)
                def _():
                    a_desc(j - 1, nxt % 3).wait()
                l_desc(nxt, nxt % 3).start()

            a_desc(j, b).start(add=True)
            return 0

        lax.fori_loop(0, nchunk, step, 0)
        for j in (nchunk - 3, nchunk - 2, nchunk - 1):
            a_desc(j, j % 3).wait()
        plsc.subcore_barrier()

        def ecopy(t, _):
            j = sid + t * NS

            @pl.when(j < NZCH)
            def _():
                sl = pl.ds(j * ZCH, ZCH)
                pltpu.sync_copy(hacc.at[sl], out_hbm.at[cid].at[sl])
            return 0

        lax.fori_loop(0, ZITER, ecopy, 0)

    buf_rows = max(chunk, ZCH)
    return pl.kernel(
        body,
        out_type=jax.ShapeDtypeStruct((NC, N, D), jnp.float32),
        mesh=_sc_mesh(),
        scratch_types=[
            pltpu.VMEM((nchunk, chunk), jnp.int32),
            pltpu.VMEM((3, buf_rows, D), jnp.float32),
            pltpu.VMEM_SHARED((N, D), jnp.float32),
            pltpu.SemaphoreType.DMA((3,)),
            pltpu.SemaphoreType.DMA((3,)),
        ],
    )


def _stats_body(xg_ref, a_ref, w1x_ref, w1a_ref, b1_ref, out_ref, e_ref):
    i = pl.program_id(0)
    a = a_ref[...]
    e_ref[...] = jnp.maximum(a, 0.0)
    xgb = xg_ref[...].astype(jnp.bfloat16)
    z = jnp.dot(xgb, w1x_ref[...], preferred_element_type=jnp.float32)
    z = z + jnp.dot(a.astype(jnp.bfloat16), w1a_ref[...],
                    preferred_element_type=jnp.float32)
    z = z + b1_ref[...]
    s1 = jnp.sum(z, axis=0, keepdims=True)
    s2 = jnp.sum(z * z, axis=0, keepdims=True)
    blk = jnp.concatenate(
        [s1, s2, jnp.zeros((6, EMB), jnp.float32)], axis=0)

    @pl.when(i == 0)
    def _():
        out_ref[...] = blk

    @pl.when(i > 0)
    def _():
        out_ref[...] = out_ref[...] + blk


def _tc_stats(xg, a, w1x, w1a, b1r):
    return pl.pallas_call(
        _stats_body,
        grid=(NEB,),
        in_specs=[
            pl.BlockSpec((EB, D), lambda i: (i, 0)),
            pl.BlockSpec((EB, DE), lambda i: (i, 0)),
            pl.BlockSpec((D, EMB), lambda i: (0, 0)),
            pl.BlockSpec((DE, EMB), lambda i: (0, 0)),
            pl.BlockSpec((1, EMB), lambda i: (0, 0)),
        ],
        out_specs=[
            pl.BlockSpec((8, EMB), lambda i: (0, 0)),
            pl.BlockSpec((EB, DE), lambda i: (i, 0)),
        ],
        out_shape=[
            jax.ShapeDtypeStruct((8, EMB), jnp.float32),
            jax.ShapeDtypeStruct((E, DE), jnp.float32),
        ],
        compiler_params=pltpu.CompilerParams(
            dimension_semantics=("arbitrary",)),
    )(xg, a, w1x, w1a, b1r)


def _main_body(st_ref, xg_ref, a_ref, w1x_ref, w1a_ref, b1_ref,
               g_ref, be_ref, w2_ref, b2_ref, msg_ref):
    st = st_ref[...]
    mu = st[0:1, :] * (1.0 / E)
    ex2 = st[1:2, :] * (1.0 / E)
    var = ex2 - mu * mu
    scale = lax.rsqrt(var + 1e-5) * g_ref[...]
    xgb = xg_ref[...].astype(jnp.bfloat16)
    z = jnp.dot(xgb, w1x_ref[...], preferred_element_type=jnp.float32)
    z = z + jnp.dot(a_ref[...].astype(jnp.bfloat16), w1a_ref[...],
                    preferred_element_type=jnp.float32)
    z = z + b1_ref[...]
    zn = (z - mu) * scale + be_ref[...]
    r = jnp.maximum(zn, 0.0).astype(jnp.bfloat16)
    msg_ref[...] = jnp.dot(r, w2_ref[...],
                           preferred_element_type=jnp.float32) + b2_ref[...]


def _tc_main(st, xg, a, w1x, w1a, b1r, gr, br, w2, b2r):
    return pl.pallas_call(
        _main_body,
        grid=(NEB,),
        in_specs=[
            pl.BlockSpec((8, EMB), lambda i: (0, 0)),
            pl.BlockSpec((EB, D), lambda i: (i, 0)),
            pl.BlockSpec((EB, DE), lambda i: (i, 0)),
            pl.BlockSpec((D, EMB), lambda i: (0, 0)),
            pl.BlockSpec((DE, EMB), lambda i: (0, 0)),
            pl.BlockSpec((1, EMB), lambda i: (0, 0)),
            pl.BlockSpec((1, EMB), lambda i: (0, 0)),
            pl.BlockSpec((1, EMB), lambda i: (0, 0)),
            pl.BlockSpec((EMB, D), lambda i: (0, 0)),
            pl.BlockSpec((1, D), lambda i: (0, 0)),
        ],
        out_specs=pl.BlockSpec((EB, D), lambda i: (i, 0)),
        out_shape=jax.ShapeDtypeStruct((E, D), jnp.float32),
        compiler_params=pltpu.CompilerParams(
            dimension_semantics=("arbitrary",)),
    )(st, xg, a, w1x, w1a, b1r, gr, br, w2, b2r)


def _final_body(p_ref, h_ref):
    h_ref[...] = jnp.maximum(p_ref[0] + p_ref[1], 0.0)


def _tc_final(partials):
    nb = 2000
    return pl.pallas_call(
        _final_body,
        grid=(N // nb,),
        in_specs=[pl.BlockSpec((NC, nb, D), lambda i: (0, i, 0))],
        out_specs=pl.BlockSpec((nb, D), lambda i: (i, 0)),
        out_shape=jax.ShapeDtypeStruct((N, D), jnp.float32),
    )(partials)


def kernel(x, edge_index, edge_attr, W1, b1, gamma, beta, W2, b2):
    src3d = edge_index[0].reshape(NW, NCHUNK, CHUNK)
    dst3d = edge_index[1].reshape(NW, NCHUNK, CHUNK)
    w1x = W1[:D]
    w1a = W1[D:]
    b1r = b1.reshape(1, EMB)
    gr = gamma.reshape(1, EMB)
    br = beta.reshape(1, EMB)
    b2r = b2.reshape(1, D)

    w1xb = w1x.astype(jnp.bfloat16)
    w1ab = w1a.astype(jnp.bfloat16)
    w2b = W2.astype(jnp.bfloat16)
    xg = _sc_gather(EPW, CHUNK, NCHUNK)(x, src3d)
    st, e_out = _tc_stats(xg, edge_attr, w1xb, w1ab, b1r)
    msg = _tc_main(st, xg, edge_attr, w1xb, w1ab, b1r, gr, br, w2b, b2r)
    partials = _sc_scatter(EPW, CHUNK, NCHUNK)(msg, dst3d)
    h = _tc_final(partials)
    return (h, e_out)


# speedup vs baseline: 1.3043x; 1.0081x over previous
import functools

import jax
import jax.numpy as jnp
from jax import lax
from jax.experimental import pallas as pl
from jax.experimental.pallas import tpu as pltpu
from jax.experimental.pallas import tpu_sc as plsc

N = 10000
E = 320000
D = 128
DE = 16
EMB = D + DE

NC = 2
NS = 16
NW = NC * NS
EPW = E // NW
CHUNK = 80
NCHUNK = EPW // CHUNK

EB = 12800
NEB = E // EB

ZCH = 80
NZCH = N // ZCH
ZITER = (NZCH + NS - 1) // NS


def _sc_mesh():
    return plsc.VectorSubcoreMesh(core_axis_name="c", subcore_axis_name="s")


@functools.cache
def _sc_gather(epw, chunk, nchunk):
    def body(x_hbm, idx_hbm, out_hbm, idx_v, rows_v, gsem, ssem):
        wid = lax.axis_index("s") * NC + lax.axis_index("c")
        pltpu.sync_copy(idx_hbm.at[wid], idx_v)
        base = wid * epw

        def g_desc(j, b):
            return pltpu.make_async_copy(
                x_hbm.at[idx_v.at[j]], rows_v.at[b], gsem.at[b])

        def s_desc(j, b):
            return pltpu.make_async_copy(
                rows_v.at[b], out_hbm.at[pl.ds(base + j * chunk, chunk)],
                ssem.at[b])

        nbuf = 5
        for k in range(nbuf - 1):
            g_desc(k, k).start()

        def step(j, _):
            b = j % nbuf
            g_desc(j, b).wait()
            nxt = j + nbuf - 1

            @pl.when(nxt < nchunk)
            def _():
                @pl.when(j >= 1)
                def _():
                    s_desc(j - 1, nxt % nbuf).wait()
                g_desc(nxt, nxt % nbuf).start()

            s_desc(j, b).start()
            return 0

        lax.fori_loop(0, nchunk, step, 0)
        for j in range(nchunk - nbuf, nchunk):
            s_desc(j, j % nbuf).wait()

    return pl.kernel(
        body,
        out_type=jax.ShapeDtypeStruct((epw * NW, D), jnp.float32),
        mesh=_sc_mesh(),
        scratch_types=[
            pltpu.VMEM((nchunk, chunk), jnp.int32),
            pltpu.VMEM((5, chunk, D), jnp.float32),
            pltpu.SemaphoreType.DMA((5,)),
            pltpu.SemaphoreType.DMA((5,)),
        ],
    )


@functools.cache
def _sc_scatter(epw, chunk, nchunk):
    def body(msg_hbm, idx_hbm, out_hbm, idx_v, rows_v, hacc, lsem, asem):
        cid = lax.axis_index("c")
        sid = lax.axis_index("s")
        wid = sid * NC + cid

        def zrow(i, _):
            def zseg(k, _):
                rows_v[0, i, pl.ds(k * 16, 16)] = jnp.zeros(
                    (16,), jnp.float32)
                return 0
            lax.fori_loop(0, D // 16, zseg, 0)
            return 0

        lax.fori_loop(0, ZCH, zrow, 0)

        def zcopy(t, _):
            j = sid + t * NS

            @pl.when(j < NZCH)
            def _():
                pltpu.sync_copy(
                    rows_v.at[0].at[pl.ds(0, ZCH)],
                    hacc.at[pl.ds(j * ZCH, ZCH)])
            return 0

        lax.fori_loop(0, ZITER, zcopy, 0)
        plsc.subcore_barrier()

        pltpu.sync_copy(idx_hbm.at[wid], idx_v)
        base = wid * epw

        def l_desc(j, b):
            return pltpu.make_async_copy(
                msg_hbm.at[pl.ds(base + j * chunk, chunk)],
                rows_v.at[b].at[pl.ds(0, chunk)], lsem.at[b])

        def a_desc(j, b):
            return pltpu.make_async_copy(
                rows_v.at[b].at[pl.ds(0, chunk)], hacc.at[idx_v.at[j]],
                asem.at[b])

        l_desc(0, 0).start()
        l_desc(1, 1).start()

        def step(j, _):
            b = j % 3
            l_desc(j, b).wait()
            nxt = j + 2

            @pl.when(nxt < nchunk)
            def _():
                @pl.when(j >= 1)
                def _():
                    a_desc(j - 1, nxt % 3).wait()
                l_desc(nxt, nxt % 3).start()

            a_desc(j, b).start(add=True)
            return 0

        lax.fori_loop(0, nchunk, step, 0)
        for j in (nchunk - 3, nchunk - 2, nchunk - 1):
            a_desc(j, j % 3).wait()
        plsc.subcore_barrier()

        def ecopy(t, _):
            j = sid + t * NS

            @pl.when(j < NZCH)
            def _():
                sl = pl.ds(j * ZCH, ZCH)
                pltpu.sync_copy(hacc.at[sl], out_hbm.at[cid].at[sl])
            return 0

        lax.fori_loop(0, ZITER, ecopy, 0)

    buf_rows = max(chunk, ZCH)
    return pl.kernel(
        body,
        out_type=jax.ShapeDtypeStruct((NC, N, D), jnp.float32),
        mesh=_sc_mesh(),
        scratch_types=[
            pltpu.VMEM((nchunk, chunk), jnp.int32),
            pltpu.VMEM((3, buf_rows, D), jnp.float32),
            pltpu.VMEM_SHARED((N, D), jnp.float32),
            pltpu.SemaphoreType.DMA((3,)),
            pltpu.SemaphoreType.DMA((3,)),
        ],
    )


def _stats_body(xg_ref, a_ref, w1x_ref, w1a_ref, b1_ref, out_ref, e_ref):
    i = pl.program_id(0)
    a = a_ref[...]
    e_ref[...] = jnp.maximum(a, 0.0)
    xgb = xg_ref[...].astype(jnp.bfloat16)
    z = jnp.dot(xgb, w1x_ref[...], preferred_element_type=jnp.float32)
    z = z + jnp.dot(a.astype(jnp.bfloat16), w1a_ref[...],
                    preferred_element_type=jnp.float32)
    z = z + b1_ref[...]
    s1 = jnp.sum(z, axis=0, keepdims=True)
    s2 = jnp.sum(z * z, axis=0, keepdims=True)
    blk = jnp.concatenate(
        [s1, s2, jnp.zeros((6, EMB), jnp.float32)], axis=0)

    @pl.when(i == 0)
    def _():
        out_ref[...] = blk

    @pl.when(i > 0)
    def _():
        out_ref[...] = out_ref[...] + blk


def _tc_stats(xg, a, w1x, w1a, b1r):
    return pl.pallas_call(
        _stats_body,
        grid=(NEB,),
        in_specs=[
            pl.BlockSpec((EB, D), lambda i: (i, 0)),
            pl.BlockSpec((EB, DE), lambda i: (i, 0)),
            pl.BlockSpec((D, EMB), lambda i: (0, 0)),
            pl.BlockSpec((DE, EMB), lambda i: (0, 0)),
            pl.BlockSpec((1, EMB), lambda i: (0, 0)),
        ],
        out_specs=[
            pl.BlockSpec((8, EMB), lambda i: (0, 0)),
            pl.BlockSpec((EB, DE), lambda i: (i, 0)),
        ],
        out_shape=[
            jax.ShapeDtypeStruct((8, EMB), jnp.float32),
            jax.ShapeDtypeStruct((E, DE), jnp.float32),
        ],
        compiler_params=pltpu.CompilerParams(
            dimension_semantics=("arbitrary",)),
    )(xg, a, w1x, w1a, b1r)


def _main_body(st_ref, xg_ref, a_ref, w1x_ref, w1a_ref, b1_ref,
               g_ref, be_ref, w2_ref, b2_ref, msg_ref):
    st = st_ref[...]
    mu = st[0:1, :] * (1.0 / E)
    ex2 = st[1:2, :] * (1.0 / E)
    var = ex2 - mu * mu
    scale = lax.rsqrt(var + 1e-5) * g_ref[...]
    xgb = xg_ref[...].astype(jnp.bfloat16)
    z = jnp.dot(xgb, w1x_ref[...], preferred_element_type=jnp.float32)
    z = z + jnp.dot(a_ref[...].astype(jnp.bfloat16), w1a_ref[...],
                    preferred_element_type=jnp.float32)
    z = z + b1_ref[...]
    zn = (z - mu) * scale + be_ref[...]
    r = jnp.maximum(zn, 0.0).astype(jnp.bfloat16)
    msg_ref[...] = jnp.dot(r, w2_ref[...],
                           preferred_element_type=jnp.float32) + b2_ref[...]


def _tc_main(st, xg, a, w1x, w1a, b1r, gr, br, w2, b2r):
    return pl.pallas_call(
        _main_body,
        grid=(NEB,),
        in_specs=[
            pl.BlockSpec((8, EMB), lambda i: (0, 0)),
            pl.BlockSpec((EB, D), lambda i: (i, 0)),
            pl.BlockSpec((EB, DE), lambda i: (i, 0)),
            pl.BlockSpec((D, EMB), lambda i: (0, 0)),
            pl.BlockSpec((DE, EMB), lambda i: (0, 0)),
            pl.BlockSpec((1, EMB), lambda i: (0, 0)),
            pl.BlockSpec((1, EMB), lambda i: (0, 0)),
            pl.BlockSpec((1, EMB), lambda i: (0, 0)),
            pl.BlockSpec((EMB, D), lambda i: (0, 0)),
            pl.BlockSpec((1, D), lambda i: (0, 0)),
        ],
        out_specs=pl.BlockSpec((EB, D), lambda i: (i, 0)),
        out_shape=jax.ShapeDtypeStruct((E, D), jnp.float32),
        compiler_params=pltpu.CompilerParams(
            dimension_semantics=("arbitrary",)),
    )(st, xg, a, w1x, w1a, b1r, gr, br, w2, b2r)


def _final_body(p_ref, h_ref):
    h_ref[...] = jnp.maximum(p_ref[0] + p_ref[1], 0.0)


def _tc_final(partials):
    nb = 2000
    return pl.pallas_call(
        _final_body,
        grid=(N // nb,),
        in_specs=[pl.BlockSpec((NC, nb, D), lambda i: (0, i, 0))],
        out_specs=pl.BlockSpec((nb, D), lambda i: (i, 0)),
        out_shape=jax.ShapeDtypeStruct((N, D), jnp.float32),
    )(partials)


def kernel(x, edge_index, edge_attr, W1, b1, gamma, beta, W2, b2):
    src3d = edge_index[0].reshape(NW, NCHUNK, CHUNK)
    dst3d = edge_index[1].reshape(NW, NCHUNK, CHUNK)
    w1x = W1[:D]
    w1a = W1[D:]
    b1r = b1.reshape(1, EMB)
    gr = gamma.reshape(1, EMB)
    br = beta.reshape(1, EMB)
    b2r = b2.reshape(1, D)

    w1xb = w1x.astype(jnp.bfloat16)
    w1ab = w1a.astype(jnp.bfloat16)
    w2b = W2.astype(jnp.bfloat16)
    xg = _sc_gather(EPW, CHUNK, NCHUNK)(x, src3d)
    st, e_out = _tc_stats(xg, edge_attr, w1xb, w1ab, b1r)
    msg = _tc_main(st, xg, edge_attr, w1xb, w1ab, b1r, gr, br, w2b, b2r)
    partials = _sc_scatter(EPW, CHUNK, NCHUNK)(msg, dst3d)
    h = _tc_final(partials)
    return (h, e_out)
